# baseline (speedup 1.0000x reference)
import functools

import jax
import jax.numpy as jnp
from jax import lax
from jax.experimental import pallas as pl
from jax.experimental.pallas import tpu as pltpu

K = 8192
D = 32
N = 4096
BETA = 0.25
DECAY = 0.99
EPS = 1e-05

KBLK = 1024


def _argmin_body(z_ref, e_ref, idx_out, best_val, best_idx):
    k = pl.program_id(0)

    @pl.when(k == 0)
    def _init():
        best_val[...] = jnp.full((N,), jnp.inf, jnp.float32)
        best_idx[...] = jnp.zeros((N,), jnp.int32)

    z = z_ref[...]
    e = e_ref[...]
    sz = jnp.sum(z * z, axis=1, keepdims=True)
    se = jnp.sum(e * e, axis=1)
    mm = lax.dot_general(z, e, (((1,), (1,)), ((), ())),
                         preferred_element_type=jnp.float32)
    dist = (sz + se[None, :]) - 2.0 * mm

    m = jnp.min(dist, axis=1)
    ids = lax.broadcasted_iota(jnp.int32, (N, KBLK), 1)
    bidx = jnp.min(jnp.where(dist == m[:, None], ids, K), axis=1) + k * KBLK

    better = m < best_val[...]
    best_val[...] = jnp.where(better, m, best_val[...])
    best_idx[...] = jnp.where(better, bidx, best_idx[...])

    @pl.when(k == pl.num_programs(0) - 1)
    def _done():
        idx_out[...] = best_idx[...]


def _argmin_indices(flat_z, embedding):
    return pl.pallas_call(
        _argmin_body,
        grid=(K // KBLK,),
        in_specs=[
            pl.BlockSpec((N, D), lambda k: (0, 0)),
            pl.BlockSpec((KBLK, D), lambda k: (k, 0)),
        ],
        out_specs=pl.BlockSpec((N,), lambda k: (0,)),
        out_shape=jax.ShapeDtypeStruct((N,), jnp.int32),
        scratch_shapes=[
            pltpu.VMEM((N,), jnp.float32),
            pltpu.VMEM((N,), jnp.int32),
        ],
    )(flat_z, embedding)


def kernel(z_e, embedding, ema_cluster_size, ema_w):
    B, Dd, H, W = z_e.shape
    flat_z = jnp.transpose(z_e, (0, 2, 3, 1)).reshape(-1, Dd)
    indices = _argmin_indices(flat_z, embedding)

    counts = jnp.zeros((K,), jnp.float32).at[indices].add(1.0)
    dw = jnp.zeros((K, D), jnp.float32).at[indices].add(flat_z)
    new_cluster_size = ema_cluster_size * DECAY + (1.0 - DECAY) * counts
    new_ema_w = ema_w * DECAY + (1.0 - DECAY) * dw
    n = new_cluster_size + EPS
    new_embedding = new_ema_w / n[:, None]
    z_q_flat = jnp.take(new_embedding, indices, axis=0)

    z_q = jnp.transpose(z_q_flat.reshape(B, H, W, Dd), (0, 3, 1, 2))
    z_q_st = z_e + (z_q - z_e)
    vq_loss = BETA * jnp.mean((z_e - z_q) ** 2)
    return (z_q_st, vq_loss, indices.reshape(B, H, W))


# speedup vs baseline: 1.1551x; 1.1551x over previous
import functools

import jax
import jax.numpy as jnp
from jax import lax
from jax.experimental import pallas as pl
from jax.experimental.pallas import tpu as pltpu
from jax.experimental.pallas import tpu_sc as plsc

K = 8192
D = 32
N = 4096
BETA = 0.25
DECAY = 0.99
EPS = 1e-05

KBLK = 1024

NW = 16
RPW = N // NW
CPW = K // NW
EPW = RPW * D
CEPW = CPW * D


def _argmin_body(z_ref, e_ref, idx_out, best_val, best_idx):
    k = pl.program_id(0)

    @pl.when(k == 0)
    def _init():
        best_val[...] = jnp.full((N,), jnp.inf, jnp.float32)
        best_idx[...] = jnp.zeros((N,), jnp.int32)

    z = z_ref[...]
    e = e_ref[...]
    sz = jnp.sum(z * z, axis=1, keepdims=True)
    se = jnp.sum(e * e, axis=1)
    mm = lax.dot_general(z, e, (((1,), (1,)), ((), ())),
                         preferred_element_type=jnp.float32)
    dist = (sz + se[None, :]) - 2.0 * mm

    m = jnp.min(dist, axis=1)
    ids = lax.broadcasted_iota(jnp.int32, (N, KBLK), 1)
    bidx = jnp.min(jnp.where(dist == m[:, None], ids, K), axis=1) + k * KBLK

    better = m < best_val[...]
    best_val[...] = jnp.where(better, m, best_val[...])
    best_idx[...] = jnp.where(better, bidx, best_idx[...])

    @pl.when(k == pl.num_programs(0) - 1)
    def _done():
        idx_out[...] = best_idx[...]


def _argmin_indices(flat_z, embedding):
    return pl.pallas_call(
        _argmin_body,
        grid=(K // KBLK,),
        in_specs=[
            pl.BlockSpec((N, D), lambda k: (0, 0)),
            pl.BlockSpec((KBLK, D), lambda k: (k, 0)),
        ],
        out_specs=pl.BlockSpec((N,), lambda k: (0,)),
        out_shape=jax.ShapeDtypeStruct((N,), jnp.int32),
        scratch_shapes=[
            pltpu.VMEM((N,), jnp.float32),
            pltpu.VMEM((N,), jnp.int32),
        ],
    )(flat_z, embedding)


def _sc_body(z_hbm, idx_hbm, cs_hbm, w_hbm, zeros_hbm, ones_hbm,
             zqst_out, loss_out,
             z_v, idx_v, eidx_v, ones_v, dw_v, w_v, cnt_v, cs_v,
             zq_v, acc_v, dw_s, cnt_s, emb_s):
    cid = lax.axis_index("c")
    sid = lax.axis_index("s")
    on = cid == 0
    w = sid
    iota16 = lax.iota(jnp.int32, 16)

    @pl.when(on)
    def _a():
        pltpu.sync_copy(z_hbm.at[pl.ds(w * EPW, EPW)], z_v)
        pltpu.sync_copy(idx_hbm.at[w], idx_v)
        pltpu.sync_copy(ones_hbm, ones_v)
        pltpu.sync_copy(zeros_hbm.at[pl.ds(w * CEPW, CEPW)],
                        dw_s.at[pl.ds(w * CEPW, CEPW)])
        pltpu.sync_copy(zeros_hbm.at[pl.ds(w * CPW, CPW)],
                        cnt_s.at[pl.ds(w * CPW, CPW)])

        for j in range(2):
            for q in range(8):
                i16 = idx_v[j, pl.ds(q * 16, 16)]
                base = (j * 8 + q) * 16 * D

                def _mk(d, _):
                    p16 = iota16 * D + (base + d)
                    plsc.store_scatter(
                        eidx_v, [p16 >> 7, p16 & 127], i16 * D + d)
                    return 0

                lax.fori_loop(0, D, _mk, 0)

    plsc.subcore_barrier()

    @pl.when(on)
    def _a2():
        for j in range(64):
            pltpu.sync_copy(z_v.at[pl.ds(j * 128, 128)],
                            dw_s.at[eidx_v.at[j]], add=True)
        for j in range(2):
            pltpu.sync_copy(ones_v, cnt_s.at[idx_v.at[j]], add=True)

    plsc.subcore_barrier()

    @pl.when(on)
    def _b():
        pltpu.sync_copy(dw_s.at[pl.ds(w * CEPW, CEPW)], dw_v)
        pltpu.sync_copy(cnt_s.at[pl.ds(w * CPW, CPW)], cnt_v)
        pltpu.sync_copy(w_hbm.at[pl.ds(w * CEPW, CEPW)], w_v)
        pltpu.sync_copy(cs_hbm.at[pl.ds(w * CPW, CPW)], cs_v)

        def chunk(c, _):
            cnt16 = cnt_v[pl.ds(c * 16, 16)]
            cs16 = cs_v[pl.ds(c * 16, 16)]
            n16 = (cs16 * DECAY + (1.0 - DECAY) * cnt16) + EPS
            codes = iota16 + c * 16

            def col(d, _):
                p16 = codes * D + d
                w16 = plsc.load_gather(w_v, [p16])
                dw16 = plsc.load_gather(dw_v, [p16])
                new16 = (w16 * DECAY + (1.0 - DECAY) * dw16) / n16
                plsc.store_scatter(dw_v, [p16], new16)
                return 0

            lax.fori_loop(0, D, col, 0)
            return 0

        lax.fori_loop(0, CPW // 16, chunk, 0)
        pltpu.sync_copy(dw_v, emb_s.at[pl.ds(w * CEPW, CEPW)])

    plsc.subcore_barrier()

    @pl.when(on)
    def _c():
        for j in range(64):
            pltpu.sync_copy(emb_s.at[eidx_v.at[j]],
                            zq_v.at[pl.ds(j * 128, 128)])

        def piece(t, acc):
            zz = z_v[pl.ds(t * 16, 16)]
            q = zq_v[pl.ds(t * 16, 16)]
            z_v[pl.ds(t * 16, 16)] = zz + (q - zz)
            dd = zz - q
            return acc + dd * dd

        acc = lax.fori_loop(0, EPW // 16, piece, jnp.zeros((16,), jnp.float32))
        acc_v[...] = acc
        pltpu.sync_copy(z_v, zqst_out.at[pl.ds(w * EPW, EPW)])
        pltpu.sync_copy(acc_v, loss_out.at[pl.ds(w * 16, 16)])


@functools.partial(
    pl.kernel,
    out_type=(
        jax.ShapeDtypeStruct((N * D,), jnp.float32),
        jax.ShapeDtypeStruct((NW * 16,), jnp.float32),
    ),
    mesh=plsc.VectorSubcoreMesh(core_axis_name="c", subcore_axis_name="s"),
    compiler_params=pltpu.CompilerParams(needs_layout_passes=False),
    scratch_types=[
        pltpu.VMEM((EPW,), jnp.float32),
        pltpu.VMEM((2, 128), jnp.int32),
        pltpu.VMEM((64, 128), jnp.int32),
        pltpu.VMEM((128,), jnp.float32),
        pltpu.VMEM((CEPW,), jnp.float32),
        pltpu.VMEM((CEPW,), jnp.float32),
        pltpu.VMEM((CPW,), jnp.float32),
        pltpu.VMEM((CPW,), jnp.float32),
        pltpu.VMEM((EPW,), jnp.float32),
        pltpu.VMEM((16,), jnp.float32),
        pltpu.VMEM_SHARED((K * D,), jnp.float32),
        pltpu.VMEM_SHARED((K,), jnp.float32),
        pltpu.VMEM_SHARED((K * D,), jnp.float32),
    ],
)
def _sc_update(z_hbm, idx_hbm, cs_hbm, w_hbm, zeros_hbm, ones_hbm,
               zqst_out, loss_out, *rest):
    _sc_body(z_hbm, idx_hbm, cs_hbm, w_hbm, zeros_hbm, ones_hbm,
             zqst_out, loss_out, *rest)


def kernel(z_e, embedding, ema_cluster_size, ema_w):
    B, Dd, H, W = z_e.shape
    flat_z = jnp.transpose(z_e, (0, 2, 3, 1)).reshape(-1, Dd)
    indices = _argmin_indices(flat_z, embedding)

    zeros = jnp.zeros((K * D,), jnp.float32)
    ones = jnp.ones((128,), jnp.float32)
    zqst_flat, loss_part = _sc_update(
        flat_z.reshape(-1), indices.reshape(NW, 2, 128), ema_cluster_size,
        ema_w.reshape(-1), zeros, ones)

    z_q_st = jnp.transpose(zqst_flat.reshape(B, H, W, Dd), (0, 3, 1, 2))
    vq_loss = BETA * (jnp.sum(loss_part) / (N * D))
    return (z_q_st, vq_loss, indices.reshape(B, H, W))


# speedup vs baseline: 1.3809x; 1.1954x over previous
import functools

import jax
import jax.numpy as jnp
from jax import lax
from jax.experimental import pallas as pl
from jax.experimental.pallas import tpu as pltpu
from jax.experimental.pallas import tpu_sc as plsc

K = 8192
D = 32
N = 4096
BETA = 0.25
DECAY = 0.99
EPS = 1e-05

KBLK = 1024

NW = 16
RPW = N // NW
CPW = K // NW
EPW = RPW * D
CEPW = CPW * D


NBLK = 1024


def _argmin_body(z_ref, e2_ref, idx_out):
    z = z_ref[...]
    e2 = e2_ref[...]
    sz = jnp.sum(z * z, axis=1, keepdims=True)
    se = 0.25 * jnp.sum(e2 * e2, axis=1)
    mm2 = lax.dot_general(z, e2, (((1,), (1,)), ((), ())),
                          preferred_element_type=jnp.float32)
    dist = (sz + se[None, :]) + mm2

    m = jnp.min(dist, axis=1)
    ids = lax.broadcasted_iota(jnp.int32, (NBLK, K), 1)
    idx_out[...] = jnp.min(jnp.where(dist == m[:, None], ids, K), axis=1)


def _argmin_indices(flat_z, embedding):
    e2 = embedding * (-2.0)
    return pl.pallas_call(
        _argmin_body,
        grid=(N // NBLK,),
        in_specs=[
            pl.BlockSpec((NBLK, D), lambda n: (n, 0)),
            pl.BlockSpec((K, D), lambda n: (0, 0)),
        ],
        out_specs=pl.BlockSpec((NBLK,), lambda n: (n,)),
        out_shape=jax.ShapeDtypeStruct((N,), jnp.int32),
    )(flat_z, e2)


def _sc_body(z_hbm, idx_hbm, cs_hbm, w_hbm, zeros_hbm, ones_hbm,
             zqst_out, loss_out,
             z_v, idx_v, eidx_v, ones_v, dw_v, w_v, cnt_v, cs_v,
             zq_v, acc_v, dw_s, cnt_s, emb_s):
    cid = lax.axis_index("c")
    sid = lax.axis_index("s")
    on = cid == 0
    w = sid
    iota16 = lax.iota(jnp.int32, 16)

    @pl.when(on)
    def _a():
        pltpu.sync_copy(z_hbm.at[pl.ds(w * EPW, EPW)], z_v)
        pltpu.sync_copy(idx_hbm.at[w], idx_v)
        pltpu.sync_copy(ones_hbm, ones_v)
        pltpu.sync_copy(zeros_hbm.at[pl.ds(w * CEPW, CEPW)],
                        dw_s.at[pl.ds(w * CEPW, CEPW)])
        pltpu.sync_copy(zeros_hbm.at[pl.ds(w * CPW, CPW)],
                        cnt_s.at[pl.ds(w * CPW, CPW)])

        for j in range(2):
            for q in range(8):
                i16 = idx_v[j, pl.ds(q * 16, 16)]
                base = (j * 8 + q) * 16 * D

                def _mk(d, _):
                    p16 = iota16 * D + (base + d)
                    plsc.store_scatter(
                        eidx_v, [p16 >> 7, p16 & 127], i16 * D + d)
                    return 0

                lax.fori_loop(0, D, _mk, 0)

    plsc.subcore_barrier()

    @pl.when(on)
    def _a2():
        for j in range(64):
            pltpu.sync_copy(z_v.at[pl.ds(j * 128, 128)],
                            dw_s.at[eidx_v.at[j]], add=True)
        for j in range(2):
            pltpu.sync_copy(ones_v, cnt_s.at[idx_v.at[j]], add=True)

    plsc.subcore_barrier()

    @pl.when(on)
    def _b():
        pltpu.sync_copy(dw_s.at[pl.ds(w * CEPW, CEPW)], dw_v)
        pltpu.sync_copy(cnt_s.at[pl.ds(w * CPW, CPW)], cnt_v)
        pltpu.sync_copy(w_hbm.at[pl.ds(w * CEPW, CEPW)], w_v)
        pltpu.sync_copy(cs_hbm.at[pl.ds(w * CPW, CPW)], cs_v)

        def chunk(c, _):
            cnt16 = cnt_v[pl.ds(c * 16, 16)]
            cs16 = cs_v[pl.ds(c * 16, 16)]
            n16 = (cs16 * DECAY + (1.0 - DECAY) * cnt16) + EPS
            codes = iota16 + c * 16

            def col(d, _):
                p16 = codes * D + d
                w16 = plsc.load_gather(w_v, [p16])
                dw16 = plsc.load_gather(dw_v, [p16])
                new16 = (w16 * DECAY + (1.0 - DECAY) * dw16) / n16
                plsc.store_scatter(dw_v, [p16], new16)
                return 0

            lax.fori_loop(0, D, col, 0)
            return 0

        lax.fori_loop(0, CPW // 16, chunk, 0)
        pltpu.sync_copy(dw_v, emb_s.at[pl.ds(w * CEPW, CEPW)])

    plsc.subcore_barrier()

    @pl.when(on)
    def _c():
        for j in range(64):
            pltpu.sync_copy(emb_s.at[eidx_v.at[j]],
                            zq_v.at[pl.ds(j * 128, 128)])

        def piece(t, acc):
            zz = z_v[pl.ds(t * 16, 16)]
            q = zq_v[pl.ds(t * 16, 16)]
            z_v[pl.ds(t * 16, 16)] = zz + (q - zz)
            dd = zz - q
            return acc + dd * dd

        acc = lax.fori_loop(0, EPW // 16, piece, jnp.zeros((16,), jnp.float32))
        acc_v[...] = acc
        pltpu.sync_copy(z_v, zqst_out.at[pl.ds(w * EPW, EPW)])
        pltpu.sync_copy(acc_v, loss_out.at[pl.ds(w * 16, 16)])


@functools.partial(
    pl.kernel,
    out_type=(
        jax.ShapeDtypeStruct((N * D,), jnp.float32),
        jax.ShapeDtypeStruct((NW * 16,), jnp.float32),
    ),
    mesh=plsc.VectorSubcoreMesh(core_axis_name="c", subcore_axis_name="s"),
    compiler_params=pltpu.CompilerParams(needs_layout_passes=False),
    scratch_types=[
        pltpu.VMEM((EPW,), jnp.float32),
        pltpu.VMEM((2, 128), jnp.int32),
        pltpu.VMEM((64, 128), jnp.int32),
        pltpu.VMEM((128,), jnp.float32),
        pltpu.VMEM((CEPW,), jnp.float32),
        pltpu.VMEM((CEPW,), jnp.float32),
        pltpu.VMEM((CPW,), jnp.float32),
        pltpu.VMEM((CPW,), jnp.float32),
        pltpu.VMEM((EPW,), jnp.float32),
        pltpu.VMEM((16,), jnp.float32),
        pltpu.VMEM_SHARED((K * D,), jnp.float32),
        pltpu.VMEM_SHARED((K,), jnp.float32),
        pltpu.VMEM_SHARED((K * D,), jnp.float32),
    ],
)
def _sc_update(z_hbm, idx_hbm, cs_hbm, w_hbm, zeros_hbm, ones_hbm,
               zqst_out, loss_out, *rest):
    _sc_body(z_hbm, idx_hbm, cs_hbm, w_hbm, zeros_hbm, ones_hbm,
             zqst_out, loss_out, *rest)


def kernel(z_e, embedding, ema_cluster_size, ema_w):
    B, Dd, H, W = z_e.shape
    flat_z = jnp.transpose(z_e, (0, 2, 3, 1)).reshape(-1, Dd)
    indices = _argmin_indices(flat_z, embedding)

    zeros = jnp.zeros((K * D,), jnp.float32)
    ones = jnp.ones((128,), jnp.float32)
    zqst_flat, loss_part = _sc_update(
        flat_z.reshape(-1), indices.reshape(NW, 2, 128), ema_cluster_size,
        ema_w.reshape(-1), zeros, ones)

    z_q_st = jnp.transpose(zqst_flat.reshape(B, H, W, Dd), (0, 3, 1, 2))
    vq_loss = BETA * (jnp.sum(loss_part) / (N * D))
    return (z_q_st, vq_loss, indices.reshape(B, H, W))


# speedup vs baseline: 1.8275x; 1.3235x over previous
import functools

import jax
import jax.numpy as jnp
from jax import lax
from jax.experimental import pallas as pl
from jax.experimental.pallas import tpu as pltpu
from jax.experimental.pallas import tpu_sc as plsc

K = 8192
D = 32
N = 4096
BETA = 0.25
DECAY = 0.99
EPS = 1e-05

KBLK = 1024

NW = 16
RPW = N // NW
CPW = K // NW
EPW = RPW * D
CEPW = CPW * D


NBLK = 1024


def _argmin_body(z_ref, e2_ref, idx_out):
    z = z_ref[...]
    e2 = e2_ref[...]
    sz = jnp.sum(z * z, axis=1, keepdims=True)
    se = 0.25 * jnp.sum(e2 * e2, axis=1)
    mm2 = lax.dot_general(z, e2, (((1,), (1,)), ((), ())),
                          preferred_element_type=jnp.float32)
    dist = (sz + se[None, :]) + mm2

    m = jnp.min(dist, axis=1)
    ids = lax.broadcasted_iota(jnp.int32, (NBLK, K), 1)
    idx_out[...] = jnp.min(jnp.where(dist == m[:, None], ids, K), axis=1)


def _argmin_indices(flat_z, embedding):
    e2 = embedding * (-2.0)
    return pl.pallas_call(
        _argmin_body,
        grid=(N // NBLK,),
        in_specs=[
            pl.BlockSpec((NBLK, D), lambda n: (n, 0)),
            pl.BlockSpec((K, D), lambda n: (0, 0)),
        ],
        out_specs=pl.BlockSpec((NBLK,), lambda n: (n,)),
        out_shape=jax.ShapeDtypeStruct((N,), jnp.int32),
    )(flat_z, e2)


def _sc_body(z_hbm, idx_hbm, cs_hbm, w_hbm, zeros_hbm, ones_hbm,
             zqst_out, loss_out,
             z_v, idx_v, eidx_v, ones_v, dw_v, w_v, cnt_v, cs_v, inv_v,
             zq_v, acc_v, dw_s, cnt_s, emb_s, sem, sem2):
    cid = lax.axis_index("c")
    sid = lax.axis_index("s")
    on = cid == 0
    w = sid
    iota16 = lax.iota(jnp.int32, 16)

    @pl.when(on)
    def _a():
        pltpu.sync_copy(idx_hbm.at[w], idx_v)
        z_dma = pltpu.async_copy(z_hbm.at[pl.ds(w * EPW, EPW)], z_v, sem)
        pltpu.async_copy(w_hbm.at[pl.ds(w * CEPW, CEPW)], w_v, sem2)
        pltpu.async_copy(cs_hbm.at[pl.ds(w * CPW, CPW)], cs_v, sem2)
        pltpu.sync_copy(zeros_hbm.at[pl.ds(w * CEPW, CEPW)],
                        dw_s.at[pl.ds(w * CEPW, CEPW)])
        pltpu.sync_copy(zeros_hbm.at[pl.ds(w * CPW, CPW)],
                        cnt_s.at[pl.ds(w * CPW, CPW)])
        pltpu.sync_copy(ones_hbm, ones_v)

        for j in range(2):
            for q in range(8):
                i16 = idx_v[j, pl.ds(q * 16, 16)]
                base = (j * 8 + q) * 16 * D

                def _mk(d, _):
                    p16 = iota16 * D + (base + d)
                    plsc.store_scatter(
                        eidx_v, [p16 >> 7, p16 & 127], i16 * D + d)
                    return 0

                lax.fori_loop(0, D, _mk, 0)
        z_dma.wait()

    plsc.subcore_barrier()

    @pl.when(on)
    def _a2():
        dmas = [pltpu.async_copy(z_v.at[pl.ds(j * 128, 128)],
                                 dw_s.at[eidx_v.at[j]], sem, add=True)
                for j in range(64)]
        dmas += [pltpu.async_copy(ones_v, cnt_s.at[idx_v.at[j]], sem,
                                  add=True)
                 for j in range(2)]
        for d in dmas:
            d.wait()

    plsc.subcore_barrier()

    @pl.when(on)
    def _b():
        pltpu.sync_copy(dw_s.at[pl.ds(w * CEPW, CEPW)], dw_v)
        pltpu.sync_copy(cnt_s.at[pl.ds(w * CPW, CPW)], cnt_v)
        pltpu.make_async_copy(w_hbm.at[pl.ds(w * CEPW, CEPW)], w_v,
                              sem2).wait()
        pltpu.make_async_copy(cs_hbm.at[pl.ds(w * CPW, CPW)], cs_v,
                              sem2).wait()

        def chunk(c, _):
            cnt16 = cnt_v[pl.ds(c * 16, 16)]
            cs16 = cs_v[pl.ds(c * 16, 16)]
            n16 = (cs16 * DECAY + (1.0 - DECAY) * cnt16) + EPS
            inv_v[pl.ds(c * 16, 16)] = 1.0 / n16
            return 0

        lax.fori_loop(0, CPW // 16, chunk, 0)

        def code(r, _):
            inv = plsc.load_gather(inv_v, [jnp.full((16,), 0, jnp.int32) + r])
            for h in range(2):
                sl = pl.ds(r * D + h * 16, 16)
                new16 = (w_v[sl] * DECAY + (1.0 - DECAY) * dw_v[sl]) * inv
                dw_v[sl] = new16
            return 0

        lax.fori_loop(0, CPW, code, 0)
        pltpu.sync_copy(dw_v, emb_s.at[pl.ds(w * CEPW, CEPW)])

    plsc.subcore_barrier()

    @pl.when(on)
    def _c():
        dmas = [pltpu.async_copy(emb_s.at[eidx_v.at[j]],
                                 zq_v.at[pl.ds(j * 128, 128)], sem)
                for j in range(64)]
        for d in dmas:
            d.wait()

        def piece(t, acc):
            zz = z_v[pl.ds(t * 16, 16)]
            q = zq_v[pl.ds(t * 16, 16)]
            z_v[pl.ds(t * 16, 16)] = zz + (q - zz)
            dd = zz - q
            return acc + dd * dd

        acc = lax.fori_loop(0, EPW // 16, piece, jnp.zeros((16,), jnp.float32))
        acc_v[...] = acc
        pltpu.sync_copy(z_v, zqst_out.at[pl.ds(w * EPW, EPW)])
        pltpu.sync_copy(acc_v, loss_out.at[pl.ds(w * 16, 16)])


@functools.partial(
    pl.kernel,
    out_type=(
        jax.ShapeDtypeStruct((N * D,), jnp.float32),
        jax.ShapeDtypeStruct((NW * 16,), jnp.float32),
    ),
    mesh=plsc.VectorSubcoreMesh(core_axis_name="c", subcore_axis_name="s"),
    compiler_params=pltpu.CompilerParams(needs_layout_passes=False),
    scratch_types=[
        pltpu.VMEM((EPW,), jnp.float32),
        pltpu.VMEM((2, 128), jnp.int32),
        pltpu.VMEM((64, 128), jnp.int32),
        pltpu.VMEM((128,), jnp.float32),
        pltpu.VMEM((CEPW,), jnp.float32),
        pltpu.VMEM((CEPW,), jnp.float32),
        pltpu.VMEM((CPW,), jnp.float32),
        pltpu.VMEM((CPW,), jnp.float32),
        pltpu.VMEM((CPW,), jnp.float32),
        pltpu.VMEM((EPW,), jnp.float32),
        pltpu.VMEM((16,), jnp.float32),
        pltpu.VMEM_SHARED((K * D,), jnp.float32),
        pltpu.VMEM_SHARED((K,), jnp.float32),
        pltpu.VMEM_SHARED((K * D,), jnp.float32),
        pltpu.SemaphoreType.DMA,
        pltpu.SemaphoreType.DMA,
    ],
)
def _sc_update(z_hbm, idx_hbm, cs_hbm, w_hbm, zeros_hbm, ones_hbm,
               zqst_out, loss_out, *rest):
    _sc_body(z_hbm, idx_hbm, cs_hbm, w_hbm, zeros_hbm, ones_hbm,
             zqst_out, loss_out, *rest)


def kernel(z_e, embedding, ema_cluster_size, ema_w):
    B, Dd, H, W = z_e.shape
    flat_z = jnp.transpose(z_e, (0, 2, 3, 1)).reshape(-1, Dd)
    indices = _argmin_indices(flat_z, embedding)

    zeros = jnp.zeros((K * D,), jnp.float32)
    ones = jnp.ones((128,), jnp.float32)
    zqst_flat, loss_part = _sc_update(
        flat_z.reshape(-1), indices.reshape(NW, 2, 128), ema_cluster_size,
        ema_w.reshape(-1), zeros, ones)

    z_q_st = jnp.transpose(zqst_flat.reshape(B, H, W, Dd), (0, 3, 1, 2))
    vq_loss = BETA * (jnp.sum(loss_part) / (N * D))
    return (z_q_st, vq_loss, indices.reshape(B, H, W))


# speedup vs baseline: 2.0682x; 1.1317x over previous
import functools

import jax
import jax.numpy as jnp
from jax import lax
from jax.experimental import pallas as pl
from jax.experimental.pallas import tpu as pltpu
from jax.experimental.pallas import tpu_sc as plsc

K = 8192
D = 32
N = 4096
BETA = 0.25
DECAY = 0.99
EPS = 1e-05

KBLK = 1024

NW = 16
RPW = N // NW
CPW = K // NW
EPW = RPW * D
CEPW = CPW * D


NBLK = 1024


def _argmin_body(z_ref, e2_ref, idx_out):
    z = z_ref[...]
    e2 = e2_ref[...]
    sz = jnp.sum(z * z, axis=1, keepdims=True)
    se = 0.25 * jnp.sum(e2 * e2, axis=1)
    mm2 = lax.dot_general(z, e2, (((1,), (1,)), ((), ())),
                          preferred_element_type=jnp.float32)
    dist = (sz + se[None, :]) + mm2

    idx_out[...] = jnp.argmin(dist, axis=1).astype(jnp.int32)


def _argmin_indices(flat_z, embedding):
    e2 = embedding * (-2.0)
    return pl.pallas_call(
        _argmin_body,
        grid=(N // NBLK,),
        in_specs=[
            pl.BlockSpec((NBLK, D), lambda n: (n, 0)),
            pl.BlockSpec((K, D), lambda n: (0, 0)),
        ],
        out_specs=pl.BlockSpec((NBLK,), lambda n: (n,)),
        out_shape=jax.ShapeDtypeStruct((N,), jnp.int32),
    )(flat_z, e2)


def _sc_body(z_hbm, idx_hbm, cs_hbm, w_hbm, zeros_hbm, ones_hbm,
             zqst_out, loss_out,
             z_v, idx_v, eidx_v, ones_v, dw_v, w_v, cnt_v, cs_v, inv_v,
             zq_v, acc_v, dw_s, cnt_s, emb_s, sem, sem2):
    cid = lax.axis_index("c")
    sid = lax.axis_index("s")
    on = cid == 0
    w = sid
    iota16 = lax.iota(jnp.int32, 16)

    @pl.when(on)
    def _a():
        pltpu.sync_copy(idx_hbm.at[w], idx_v)
        z_dma = pltpu.async_copy(z_hbm.at[pl.ds(w * EPW, EPW)], z_v, sem)
        pltpu.async_copy(w_hbm.at[pl.ds(w * CEPW, CEPW)], w_v, sem2)
        pltpu.async_copy(cs_hbm.at[pl.ds(w * CPW, CPW)], cs_v, sem2)
        pltpu.sync_copy(zeros_hbm.at[pl.ds(w * CEPW, CEPW)],
                        dw_s.at[pl.ds(w * CEPW, CEPW)])
        pltpu.sync_copy(zeros_hbm.at[pl.ds(w * CPW, CPW)],
                        cnt_s.at[pl.ds(w * CPW, CPW)])
        pltpu.sync_copy(ones_hbm, ones_v)

        for j in range(2):
            for q in range(8):
                i16 = idx_v[j, pl.ds(q * 16, 16)]
                base = (j * 8 + q) * 16 * D

                def _mk(d, _):
                    p16 = iota16 * D + (base + d)
                    plsc.store_scatter(
                        eidx_v, [p16 >> 7, p16 & 127], i16 * D + d)
                    return 0

                lax.fori_loop(0, D, _mk, 0)
        z_dma.wait()

    plsc.subcore_barrier()

    @pl.when(on)
    def _a2():
        dmas = [pltpu.async_copy(z_v.at[pl.ds(j * 128, 128)],
                                 dw_s.at[eidx_v.at[j]], sem, add=True)
                for j in range(64)]
        dmas += [pltpu.async_copy(ones_v, cnt_s.at[idx_v.at[j]], sem,
                                  add=True)
                 for j in range(2)]
        for d in dmas:
            d.wait()

    plsc.subcore_barrier()

    @pl.when(on)
    def _b():
        pltpu.sync_copy(dw_s.at[pl.ds(w * CEPW, CEPW)], dw_v)
        pltpu.sync_copy(cnt_s.at[pl.ds(w * CPW, CPW)], cnt_v)
        pltpu.make_async_copy(w_hbm.at[pl.ds(w * CEPW, CEPW)], w_v,
                              sem2).wait()
        pltpu.make_async_copy(cs_hbm.at[pl.ds(w * CPW, CPW)], cs_v,
                              sem2).wait()

        def chunk(c, _):
            cnt16 = cnt_v[pl.ds(c * 16, 16)]
            cs16 = cs_v[pl.ds(c * 16, 16)]
            n16 = (cs16 * DECAY + (1.0 - DECAY) * cnt16) + EPS
            inv_v[pl.ds(c * 16, 16)] = 1.0 / n16
            return 0

        lax.fori_loop(0, CPW // 16, chunk, 0)

        def code(r, _):
            inv = plsc.load_gather(inv_v, [jnp.full((16,), 0, jnp.int32) + r])
            for h in range(2):
                sl = pl.ds(r * D + h * 16, 16)
                new16 = (w_v[sl] * DECAY + (1.0 - DECAY) * dw_v[sl]) * inv
                dw_v[sl] = new16
            return 0

        lax.fori_loop(0, CPW, code, 0)
        pltpu.sync_copy(dw_v, emb_s.at[pl.ds(w * CEPW, CEPW)])

    plsc.subcore_barrier()

    @pl.when(on)
    def _c():
        dmas = [pltpu.async_copy(emb_s.at[eidx_v.at[j]],
                                 zq_v.at[pl.ds(j * 128, 128)], sem)
                for j in range(64)]
        for d in dmas:
            d.wait()

        def piece(t, acc):
            zz = z_v[pl.ds(t * 16, 16)]
            q = zq_v[pl.ds(t * 16, 16)]
            z_v[pl.ds(t * 16, 16)] = zz + (q - zz)
            dd = zz - q
            return acc + dd * dd

        acc = lax.fori_loop(0, EPW // 16, piece, jnp.zeros((16,), jnp.float32))
        acc_v[...] = acc
        pltpu.sync_copy(z_v, zqst_out.at[pl.ds(w * EPW, EPW)])
        pltpu.sync_copy(acc_v, loss_out.at[pl.ds(w * 16, 16)])


@functools.partial(
    pl.kernel,
    out_type=(
        jax.ShapeDtypeStruct((N * D,), jnp.float32),
        jax.ShapeDtypeStruct((NW * 16,), jnp.float32),
    ),
    mesh=plsc.VectorSubcoreMesh(core_axis_name="c", subcore_axis_name="s"),
    compiler_params=pltpu.CompilerParams(needs_layout_passes=False),
    scratch_types=[
        pltpu.VMEM((EPW,), jnp.float32),
        pltpu.VMEM((2, 128), jnp.int32),
        pltpu.VMEM((64, 128), jnp.int32),
        pltpu.VMEM((128,), jnp.float32),
        pltpu.VMEM((CEPW,), jnp.float32),
        pltpu.VMEM((CEPW,), jnp.float32),
        pltpu.VMEM((CPW,), jnp.float32),
        pltpu.VMEM((CPW,), jnp.float32),
        pltpu.VMEM((CPW,), jnp.float32),
        pltpu.VMEM((EPW,), jnp.float32),
        pltpu.VMEM((16,), jnp.float32),
        pltpu.VMEM_SHARED((K * D,), jnp.float32),
        pltpu.VMEM_SHARED((K,), jnp.float32),
        pltpu.VMEM_SHARED((K * D,), jnp.float32),
        pltpu.SemaphoreType.DMA,
        pltpu.SemaphoreType.DMA,
    ],
)
def _sc_update(z_hbm, idx_hbm, cs_hbm, w_hbm, zeros_hbm, ones_hbm,
               zqst_out, loss_out, *rest):
    _sc_body(z_hbm, idx_hbm, cs_hbm, w_hbm, zeros_hbm, ones_hbm,
             zqst_out, loss_out, *rest)


def kernel(z_e, embedding, ema_cluster_size, ema_w):
    B, Dd, H, W = z_e.shape
    flat_z = jnp.transpose(z_e, (0, 2, 3, 1)).reshape(-1, Dd)
    indices = _argmin_indices(flat_z, embedding)

    zeros = jnp.zeros((K * D,), jnp.float32)
    ones = jnp.ones((128,), jnp.float32)
    zqst_flat, loss_part = _sc_update(
        flat_z.reshape(-1), indices.reshape(NW, 2, 128), ema_cluster_size,
        ema_w.reshape(-1), zeros, ones)

    z_q_st = jnp.transpose(zqst_flat.reshape(B, H, W, Dd), (0, 3, 1, 2))
    vq_loss = BETA * (jnp.sum(loss_part) / (N * D))
    return (z_q_st, vq_loss, indices.reshape(B, H, W))
